# baseline (speedup 1.0000x reference)
import functools

import jax
import jax.numpy as jnp
from jax import lax
from jax.experimental import pallas as pl
from jax.experimental.pallas import tpu as pltpu
from jax.experimental.pallas import tpu_sc as plsc

_B, _CTX, _D, _V = 1024, 20, 64, 100000
_DP = 128
_NC, _NS, _L = 2, 16, 16
_NW = _NC * _NS
_BPW = _B // _NW
_IPW = _BPW * _CTX
_DCHUNKS = _D // _L


@functools.cache
def _gather_sum_fn():
    mesh = plsc.VectorSubcoreMesh(
        core_axis_name="c", subcore_axis_name="s",
        num_cores=_NC, num_subcores=_NS)

    @functools.partial(
        pl.kernel,
        out_type=jax.ShapeDtypeStruct((_B, _DP), jnp.float32),
        mesh=mesh,
        scratch_types=[
            pltpu.VMEM((_IPW,), jnp.int32),
            pltpu.VMEM((_IPW, _DP), jnp.float32),
            pltpu.VMEM((_BPW, _DP), jnp.float32),
            pltpu.SemaphoreType.DMA,
        ],
        compiler_params=pltpu.CompilerParams(use_tc_tiling_on_sc=False),
    )
    def _gather_sum(idx_hbm, table_hbm, out_hbm, idx_v, rows_v, acc_v, sem):
        wid = lax.axis_index("s") * _NC + lax.axis_index("c")
        base = wid * _IPW
        pltpu.sync_copy(idx_hbm.at[pl.ds(base, _IPW)], idx_v)
        pltpu.async_copy(table_hbm.at[idx_v], rows_v, sem).wait()

        def row_body(b, carry):
            def ctx_body(c, accs):
                r = b * _CTX + c
                return tuple(accs[k] + rows_v[r, pl.ds(k * _L, _L)]
                             for k in range(_DCHUNKS))

            accs = lax.fori_loop(
                0, _CTX, ctx_body,
                tuple(jnp.zeros((_L,), jnp.float32) for _ in range(_DCHUNKS)))
            for k in range(_DCHUNKS):
                acc_v[b, pl.ds(k * _L, _L)] = accs[k]
            return carry

        lax.fori_loop(0, _BPW, row_body, 0)
        pltpu.sync_copy(acc_v, out_hbm.at[pl.ds(wid * _BPW, _BPW)])

    return _gather_sum


_BVR = 4096
_NMAIN = _V // _BVR
_VMAIN = _NMAIN * _BVR
_VTAIL = _V - _VMAIN
_BTAIL = 32
_NTAIL = _VTAIL // _BTAIL


def _mm_main_body(wt_ref, x_ref, o_ref):
    w = wt_ref[...].astype(jnp.bfloat16)
    x = x_ref[:, : _D].astype(jnp.bfloat16)
    o_ref[...] = lax.dot_general(
        w, x, (((0,), (1,)), ((), ())), preferred_element_type=jnp.float32)


def _mm_tail_body(w_ref, x_ref, _, o_ref):
    w = w_ref[...].astype(jnp.bfloat16)
    x = x_ref[:, : _D].astype(jnp.bfloat16)
    o_ref[...] = lax.dot_general(
        w, x, (((1,), (1,)), ((), ())), preferred_element_type=jnp.float32)


def _scores_t(out_embed, sum_embeds):
    main = pl.pallas_call(
        _mm_main_body,
        grid=(_NMAIN,),
        in_specs=[
            pl.BlockSpec((_D, _BVR), lambda i: (0, i)),
            pl.BlockSpec((_B, _DP), lambda i: (0, 0)),
        ],
        out_specs=pl.BlockSpec((_BVR, _B), lambda i: (i, 0)),
        out_shape=jax.ShapeDtypeStruct((_V, _B), jnp.float32),
        compiler_params=pltpu.CompilerParams(
            dimension_semantics=("parallel",)),
    )(out_embed.T, sum_embeds)

    w_tail = lax.slice(out_embed, (_VMAIN, 0), (_V, _D))
    return pl.pallas_call(
        _mm_tail_body,
        grid=(_NTAIL,),
        in_specs=[
            pl.BlockSpec((_BTAIL, _D), lambda i: (i, 0)),
            pl.BlockSpec((_B, _DP), lambda i: (0, 0)),
            pl.BlockSpec(memory_space=pltpu.HBM),
        ],
        out_specs=pl.BlockSpec(
            (_BTAIL, _B), lambda i: (_VMAIN // _BTAIL + i, 0)),
        out_shape=jax.ShapeDtypeStruct((_V, _B), jnp.float32),
        input_output_aliases={2: 0},
    )(w_tail, sum_embeds, main)


def kernel(context, in_embed, out_embed):
    table = jnp.concatenate([in_embed, in_embed], axis=1)
    sum_embeds = _gather_sum_fn()(context.reshape(-1), table)
    return _scores_t(out_embed, sum_embeds).T


# speedup vs baseline: 1.0450x; 1.0450x over previous
import functools

import jax
import jax.numpy as jnp
from jax import lax
from jax.experimental import pallas as pl
from jax.experimental.pallas import tpu as pltpu
from jax.experimental.pallas import tpu_sc as plsc

_B, _CTX, _D, _V = 1024, 20, 64, 100000
_DP = 128
_NC, _NS, _L = 2, 16, 16
_NW = _NC * _NS
_BPW = _B // _NW
_IPW = _BPW * _CTX
_DCHUNKS = _D // _L


@functools.cache
def _gather_sum_fn():
    mesh = plsc.VectorSubcoreMesh(
        core_axis_name="c", subcore_axis_name="s",
        num_cores=_NC, num_subcores=_NS)

    @functools.partial(
        pl.kernel,
        out_type=jax.ShapeDtypeStruct((_B, _DP), jnp.float32),
        mesh=mesh,
        scratch_types=[
            pltpu.VMEM((_IPW,), jnp.int32),
            pltpu.VMEM((_IPW, _D), jnp.float32),
            pltpu.VMEM((_BPW, _DP), jnp.float32),
            pltpu.SemaphoreType.DMA,
        ],
        compiler_params=pltpu.CompilerParams(use_tc_tiling_on_sc=False),
    )
    def _gather_sum(idx_hbm, table_hbm, out_hbm, idx_v, rows_v, acc_v, sem):
        wid = lax.axis_index("s") * _NC + lax.axis_index("c")
        base = wid * _IPW
        pltpu.sync_copy(idx_hbm.at[pl.ds(base, _IPW)], idx_v)
        pltpu.async_copy(table_hbm.at[idx_v], rows_v, sem).wait()

        def row_body(b, carry):
            def ctx_body(c, accs):
                r = b * _CTX + c
                return tuple(accs[k] + rows_v[r, pl.ds(k * _L, _L)]
                             for k in range(_DCHUNKS))

            accs = lax.fori_loop(
                0, _CTX, ctx_body,
                tuple(jnp.zeros((_L,), jnp.float32) for _ in range(_DCHUNKS)))
            for k in range(_DCHUNKS):
                acc_v[b, pl.ds(k * _L, _L)] = accs[k]
            return carry

        lax.fori_loop(0, _BPW, row_body, 0)
        pltpu.sync_copy(acc_v, out_hbm.at[pl.ds(wid * _BPW, _BPW)])

    return _gather_sum


_BVR = 4096
_NMAIN = _V // _BVR
_VMAIN = _NMAIN * _BVR
_VTAIL = _V - _VMAIN
_BTAIL = 32
_NTAIL = _VTAIL // _BTAIL


def _mm_main_body(wt_ref, x_ref, o_ref):
    w = wt_ref[...].astype(jnp.bfloat16)
    x = x_ref[:, : _D].astype(jnp.bfloat16)
    o_ref[...] = lax.dot_general(
        w, x, (((0,), (1,)), ((), ())), preferred_element_type=jnp.float32)


def _mm_tail_body(w_ref, x_ref, _, o_ref):
    w = w_ref[...].astype(jnp.bfloat16)
    x = x_ref[:, : _D].astype(jnp.bfloat16)
    o_ref[...] = lax.dot_general(
        w, x, (((1,), (1,)), ((), ())), preferred_element_type=jnp.float32)


def _scores_t(out_embed, sum_embeds):
    main = pl.pallas_call(
        _mm_main_body,
        grid=(_NMAIN,),
        in_specs=[
            pl.BlockSpec((_D, _BVR), lambda i: (0, i)),
            pl.BlockSpec((_B, _DP), lambda i: (0, 0)),
        ],
        out_specs=pl.BlockSpec((_BVR, _B), lambda i: (i, 0)),
        out_shape=jax.ShapeDtypeStruct((_V, _B), jnp.float32),
        compiler_params=pltpu.CompilerParams(
            dimension_semantics=("parallel",)),
    )(out_embed.T, sum_embeds)

    w_tail = lax.slice(out_embed, (_VMAIN, 0), (_V, _D))
    return pl.pallas_call(
        _mm_tail_body,
        grid=(_NTAIL,),
        in_specs=[
            pl.BlockSpec((_BTAIL, _D), lambda i: (i, 0)),
            pl.BlockSpec((_B, _DP), lambda i: (0, 0)),
            pl.BlockSpec(memory_space=pltpu.HBM),
        ],
        out_specs=pl.BlockSpec(
            (_BTAIL, _B), lambda i: (_VMAIN // _BTAIL + i, 0)),
        out_shape=jax.ShapeDtypeStruct((_V, _B), jnp.float32),
        input_output_aliases={2: 0},
    )(w_tail, sum_embeds, main)


def kernel(context, in_embed, out_embed):
    sum_embeds = _gather_sum_fn()(context.reshape(-1), in_embed)
    return _scores_t(out_embed, sum_embeds).T


# speedup vs baseline: 1.0711x; 1.0249x over previous
import functools

import jax
import jax.numpy as jnp
from jax import lax
from jax.experimental import pallas as pl
from jax.experimental.pallas import tpu as pltpu
from jax.experimental.pallas import tpu_sc as plsc

_B, _CTX, _D, _V = 1024, 20, 64, 100000
_DP = 128
_NC, _NS, _L = 2, 16, 16
_NW = _NC * _NS
_BPW = _B // _NW
_IPW = _BPW * _CTX
_DCHUNKS = _D // _L


@functools.cache
def _gather_sum_fn():
    mesh = plsc.VectorSubcoreMesh(
        core_axis_name="c", subcore_axis_name="s",
        num_cores=_NC, num_subcores=_NS)

    @functools.partial(
        pl.kernel,
        out_type=jax.ShapeDtypeStruct((_B, _DP), jnp.float32),
        mesh=mesh,
        scratch_types=[
            pltpu.VMEM((_IPW,), jnp.int32),
            pltpu.VMEM((_IPW, _DP), jnp.float32),
            pltpu.VMEM((_BPW, _DP), jnp.float32),
            pltpu.SemaphoreType.DMA,
        ],
        compiler_params=pltpu.CompilerParams(use_tc_tiling_on_sc=False),
    )
    def _gather_sum(idx_hbm, table_hbm, out_hbm, idx_v, rows_v, acc_v, sem):
        wid = lax.axis_index("s") * _NC + lax.axis_index("c")
        base = wid * _IPW
        pltpu.sync_copy(idx_hbm.at[pl.ds(base, _IPW)], idx_v)
        pltpu.async_copy(table_hbm.at[idx_v], rows_v, sem).wait()

        def row_body(b, carry):
            def ctx_body(c, accs):
                r = b * _CTX + c
                return tuple(accs[k] + rows_v[r, pl.ds(k * _L, _L)]
                             for k in range(_DCHUNKS))

            accs = lax.fori_loop(
                0, _CTX, ctx_body,
                tuple(jnp.zeros((_L,), jnp.float32) for _ in range(_DCHUNKS)))
            for k in range(_DCHUNKS):
                acc_v[b, pl.ds(k * _L, _L)] = accs[k]
            return carry

        lax.fori_loop(0, _BPW, row_body, 0)
        pltpu.sync_copy(acc_v, out_hbm.at[pl.ds(wid * _BPW, _BPW)])

    return _gather_sum


_BVR = 4096
_NMAIN = _V // _BVR
_VMAIN = _NMAIN * _BVR
_VTAIL = _V - _VMAIN
_BTAIL = 32
_NTAIL = _VTAIL // _BTAIL


def _mm_main_body(wt_ref, x_ref, o_ref):
    w = wt_ref[...].astype(jnp.bfloat16)
    x = x_ref[:, : _D].astype(jnp.bfloat16)
    o_ref[...] = lax.dot_general(
        w, x, (((0,), (1,)), ((), ())), preferred_element_type=jnp.float32)


def _mm_tail_body(w_ref, x_ref, _, o_ref):
    w = w_ref[...].astype(jnp.bfloat16)
    x = x_ref[:, : _D].astype(jnp.bfloat16)
    o_ref[...] = lax.dot_general(
        w, x, (((1,), (1,)), ((), ())), preferred_element_type=jnp.float32)


def _scores_t(out_embed, sum_embeds):
    main = pl.pallas_call(
        _mm_main_body,
        grid=(_NMAIN,),
        in_specs=[
            pl.BlockSpec((_D, _BVR), lambda i: (0, i)),
            pl.BlockSpec((_B, _DP), lambda i: (0, 0)),
        ],
        out_specs=pl.BlockSpec((_BVR, _B), lambda i: (i, 0)),
        out_shape=jax.ShapeDtypeStruct((_V, _B), jnp.float32),
        compiler_params=pltpu.CompilerParams(
            dimension_semantics=("parallel",)),
    )(out_embed.T, sum_embeds)

    w_tail = lax.slice(out_embed, (_VMAIN, 0), (_V, _D))
    return pl.pallas_call(
        _mm_tail_body,
        grid=(_NTAIL,),
        in_specs=[
            pl.BlockSpec((_BTAIL, _D), lambda i: (i, 0)),
            pl.BlockSpec((_B, _DP), lambda i: (0, 0)),
            pl.BlockSpec(memory_space=pltpu.HBM),
        ],
        out_specs=pl.BlockSpec(
            (_BTAIL, _B), lambda i: (_VMAIN // _BTAIL + i, 0)),
        out_shape=jax.ShapeDtypeStruct((_V, _B), jnp.float32),
        input_output_aliases={2: 0},
    )(w_tail, sum_embeds, main)


def kernel(context, in_embed, out_embed):
    table = jnp.pad(in_embed, ((0, 0), (0, _DP - _D)))
    sum_embeds = _gather_sum_fn()(context.reshape(-1), table)
    return _scores_t(out_embed, sum_embeds).T
